# baseline (speedup 1.0000x reference)
import functools

import jax
import jax.numpy as jnp
from jax import lax
from jax.experimental import pallas as pl
from jax.experimental.pallas import tpu as pltpu
from jax.experimental.pallas import tpu_sc as plsc

D = 128
T = 200
CHUNK = 128
NBUF = 5


@functools.lru_cache(maxsize=None)
def _build(n_rows: int):
    info = plsc.get_sparse_core_info()
    nw = info.num_cores * info.num_subcores
    rows_per_w = n_rows // nw
    n_chunks = rows_per_w // CHUNK
    assert n_rows == nw * n_chunks * CHUNK
    assert n_chunks >= 2 * NBUF and (n_chunks - NBUF) % NBUF == 0
    mesh = plsc.VectorSubcoreMesh(core_axis_name="c", subcore_axis_name="s")

    @functools.partial(
        pl.kernel,
        mesh=mesh,
        out_type=jax.ShapeDtypeStruct((n_rows, D), jnp.float32),
        scratch_types=[
            pltpu.VMEM((NBUF, CHUNK), jnp.int32),
            *([pltpu.VMEM((CHUNK, D), jnp.float32)] * NBUF),
            pltpu.VMEM_SHARED((2 * T, D), jnp.float32),
            *([pltpu.SemaphoreType.DMA] * (4 * NBUF)),
        ],
    )
    def k(ids_hbm, tok_hbm, pos_hbm, out_hbm, idx_v, *bufs):
        rows = bufs[:NBUF]
        pos_v = bufs[NBUF]
        psem = bufs[NBUF + 1:2 * NBUF + 1]
        gsem = bufs[2 * NBUF + 1:3 * NBUF + 1]
        g2sem = bufs[3 * NBUF + 1:4 * NBUF + 1]
        ssem = bufs[4 * NBUF + 1:]
        wid = lax.axis_index("s") * info.num_cores + lax.axis_index("c")
        base = wid * rows_per_w

        def start_prefill(c, b):
            po = lax.rem((wid + nw * c) * CHUNK, T)
            pltpu.async_copy(pos_v.at[pl.ds(po, CHUNK)], rows[b], psem[b])

        def wait_prefill(b):
            pltpu.make_async_copy(pos_v.at[pl.ds(0, CHUNK)], rows[b],
                                  psem[b]).wait()

        H = CHUNK // 2

        def start_gather(c, b):
            pltpu.sync_copy(ids_hbm.at[pl.ds((wid + nw * c) * CHUNK, CHUNK)],
                            idx_v.at[b])
            pltpu.async_copy(tok_hbm.at[idx_v.at[b].at[pl.ds(0, H)]],
                             rows[b].at[pl.ds(0, H)], gsem[b], add=True)
            pltpu.async_copy(tok_hbm.at[idx_v.at[b].at[pl.ds(H, H)]],
                             rows[b].at[pl.ds(H, H)], g2sem[b], add=True)

        def wait_gather(b):
            pltpu.make_async_copy(tok_hbm.at[idx_v.at[b].at[pl.ds(0, H)]],
                                  rows[b].at[pl.ds(0, H)], gsem[b]).wait()
            pltpu.make_async_copy(tok_hbm.at[idx_v.at[b].at[pl.ds(H, H)]],
                                  rows[b].at[pl.ds(H, H)], g2sem[b]).wait()

        def start_scatter(c, b):
            pltpu.async_copy(rows[b],
                             out_hbm.at[pl.ds((wid + nw * c) * CHUNK,
                                              CHUNK)], ssem[b])

        def wait_scatter(b):
            pltpu.make_async_copy(rows[b], out_hbm.at[pl.ds(0, CHUNK)],
                                  ssem[b]).wait()

        @pl.when(lax.axis_index("s") == 0)
        def _stage_pos():
            pltpu.sync_copy(pos_hbm, pos_v.at[pl.ds(0, T)])
            pltpu.sync_copy(pos_hbm, pos_v.at[pl.ds(T, T)])

        plsc.subcore_barrier()

        for j in range(NBUF - 1):
            start_prefill(j, j)
        for j in range(NBUF - 2):
            wait_prefill(j)
            start_gather(j, j)

        wait_gather(0)
        start_scatter(0, 0)
        start_prefill(NBUF - 1, NBUF - 1)
        wait_prefill(NBUF - 2)
        start_gather(NBUF - 2, NBUF - 2)

        def group(i, _):
            c0 = 1 + i * NBUF
            for j in range(NBUF):
                b = (1 + j) % NBUF
                c = c0 + j
                wait_gather(b)
                start_scatter(c, b)
                wait_scatter((b + NBUF - 1) % NBUF)
                start_prefill(c + NBUF - 1, (b + NBUF - 1) % NBUF)
                wait_prefill((b + NBUF - 2) % NBUF)
                start_gather(c + NBUF - 2, (b + NBUF - 2) % NBUF)
            return 0

        lax.fori_loop(0, (n_chunks - NBUF) // NBUF, group, 0)

        c = n_chunks - (NBUF - 1)
        b = c % NBUF
        wait_gather(b)
        start_scatter(c, b)
        wait_scatter((b + NBUF - 1) % NBUF)
        wait_prefill((b + NBUF - 2) % NBUF)
        start_gather(c + NBUF - 2, (b + NBUF - 2) % NBUF)
        for j in range(1, NBUF - 1):
            c = n_chunks - (NBUF - 1) + j
            b = c % NBUF
            wait_gather(b)
            start_scatter(c, b)
            wait_scatter((b + NBUF - 1) % NBUF)

        wait_scatter((n_chunks - 1) % NBUF)

    return k


def kernel(input_ids, tok_table, pos_table):
    b, t = input_ids.shape
    ids = input_ids.reshape(-1).astype(jnp.int32)
    out = _build(b * t)(ids, tok_table, pos_table)
    return out.reshape(b, t, D)


# speedup vs baseline: 1.0037x; 1.0037x over previous
import functools

import jax
import jax.numpy as jnp
from jax import lax
from jax.experimental import pallas as pl
from jax.experimental.pallas import tpu as pltpu
from jax.experimental.pallas import tpu_sc as plsc

D = 128
T = 200
CHUNK = 128
NBUF = 5


@functools.lru_cache(maxsize=None)
def _build(n_rows: int):
    info = plsc.get_sparse_core_info()
    nw = info.num_cores * info.num_subcores
    rows_per_w = n_rows // nw
    n_chunks = rows_per_w // CHUNK
    assert n_rows == nw * n_chunks * CHUNK
    assert n_chunks >= 2 * NBUF and (n_chunks - NBUF) % NBUF == 0
    mesh = plsc.VectorSubcoreMesh(core_axis_name="c", subcore_axis_name="s")

    @functools.partial(
        pl.kernel,
        mesh=mesh,
        out_type=jax.ShapeDtypeStruct((n_rows, D), jnp.float32),
        scratch_types=[
            pltpu.VMEM((NBUF, CHUNK), jnp.int32),
            *([pltpu.VMEM((CHUNK, D), jnp.float32)] * NBUF),
            pltpu.VMEM_SHARED((2 * T, D), jnp.float32),
            *([pltpu.SemaphoreType.DMA] * (3 * NBUF)),
        ],
    )
    def k(ids_hbm, tok_hbm, pos_hbm, out_hbm, idx_v, *bufs):
        rows = bufs[:NBUF]
        pos_v = bufs[NBUF]
        psem = bufs[NBUF + 1:2 * NBUF + 1]
        gsem = bufs[2 * NBUF + 1:3 * NBUF + 1]
        ssem = bufs[3 * NBUF + 1:]
        wid = lax.axis_index("s") * info.num_cores + lax.axis_index("c")
        base = wid * rows_per_w

        def start_prefill(c, b):
            po = lax.rem((wid + nw * c) * CHUNK, T)
            pltpu.async_copy(pos_v.at[pl.ds(po, CHUNK)], rows[b], psem[b])

        def wait_prefill(b):
            pltpu.make_async_copy(pos_v.at[pl.ds(0, CHUNK)], rows[b],
                                  psem[b]).wait()

        def start_gather(c, b):
            pltpu.sync_copy(ids_hbm.at[pl.ds((wid + nw * c) * CHUNK, CHUNK)],
                            idx_v.at[b])
            pltpu.async_copy(tok_hbm.at[idx_v.at[b]], rows[b], gsem[b],
                             add=True)

        def wait_gather(b):
            pltpu.make_async_copy(tok_hbm.at[idx_v.at[b]], rows[b],
                                  gsem[b]).wait()

        def start_scatter(c, b):
            pltpu.async_copy(rows[b],
                             out_hbm.at[pl.ds((wid + nw * c) * CHUNK,
                                              CHUNK)], ssem[b])

        def wait_scatter(b):
            pltpu.make_async_copy(rows[b], out_hbm.at[pl.ds(0, CHUNK)],
                                  ssem[b]).wait()

        @pl.when(lax.axis_index("s") == 0)
        def _stage_pos():
            pltpu.sync_copy(pos_hbm, pos_v.at[pl.ds(0, T)])
            pltpu.sync_copy(pos_hbm, pos_v.at[pl.ds(T, T)])

        plsc.subcore_barrier()

        for j in range(NBUF - 1):
            start_prefill(j, j)
        for j in range(NBUF - 2):
            wait_prefill(j)
            start_gather(j, j)

        wait_gather(0)
        start_scatter(0, 0)
        start_prefill(NBUF - 1, NBUF - 1)
        wait_prefill(NBUF - 2)
        start_gather(NBUF - 2, NBUF - 2)

        def group(i, _):
            c0 = 1 + i * NBUF
            for j in range(NBUF):
                b = (1 + j) % NBUF
                c = c0 + j
                wait_gather(b)
                start_scatter(c, b)
                wait_scatter((b + NBUF - 1) % NBUF)
                start_prefill(c + NBUF - 1, (b + NBUF - 1) % NBUF)
                wait_prefill((b + NBUF - 2) % NBUF)
                start_gather(c + NBUF - 2, (b + NBUF - 2) % NBUF)
            return 0

        lax.fori_loop(0, (n_chunks - NBUF) // NBUF, group, 0)

        c = n_chunks - (NBUF - 1)
        b = c % NBUF
        wait_gather(b)
        start_scatter(c, b)
        wait_scatter((b + NBUF - 1) % NBUF)
        wait_prefill((b + NBUF - 2) % NBUF)
        start_gather(c + NBUF - 2, (b + NBUF - 2) % NBUF)
        for j in range(1, NBUF - 1):
            c = n_chunks - (NBUF - 1) + j
            b = c % NBUF
            wait_gather(b)
            start_scatter(c, b)
            wait_scatter((b + NBUF - 1) % NBUF)

        wait_scatter((n_chunks - 1) % NBUF)

    return k


def kernel(input_ids, tok_table, pos_table):
    b, t = input_ids.shape
    ids = input_ids.reshape(-1).astype(jnp.int32)
    out = _build(b * t)(ids, tok_table, pos_table)
    return out.reshape(b, t, D)
